# baseline (speedup 1.0000x reference)
import jax
import jax.numpy as jnp
from jax import lax
from jax.experimental import pallas as pl
from jax.experimental.pallas import tpu as pltpu
from jax.experimental.pallas import tpu_sc as plsc

_N_FIELDS = 26
_VOCAB = 100000
_EMBED = 32
_BATCH = 16384

_NC = 2
_NS = 16
_L = 16
_PS = 25600
_SIZES = (_PS, _PS, _PS, _VOCAB - 3 * _PS)


def _embed_body(xt_hbm, tbl_hbm, out_hbm, buf0, buf1, buf3, idx_v, out_v,
                sem_a, sem_b):
    c = lax.axis_index("c")
    s = lax.axis_index("s")
    e = s * _NC + c

    @plsc.parallel_loop(0, _BATCH, _L, unroll=8)
    def _zero(o):
        out_v[pl.ds(o, _L)] = jnp.zeros((_L,), jnp.float32)

    bufs = (buf0, buf1, buf0, buf3)
    sems = (sem_a, sem_b, sem_a, sem_b)

    def fire(i, p):
        pltpu.async_copy(
            tbl_hbm.at[i * _EMBED + e, pl.ds(p * _PS, _SIZES[p])],
            bufs[p], sems[p])

    def wait(p):
        pltpu.make_async_copy(
            tbl_hbm.at[0, pl.ds(p * _PS, _SIZES[p])],
            bufs[p], sems[p]).wait()

    def gather_pass(p):
        base = p * _PS
        size = _SIZES[p]
        buf = bufs[p]

        @plsc.parallel_loop(0, _BATCH, _L, unroll=8)
        def _gather(o):
            off = idx_v[pl.ds(o, _L)] - base
            m = plsc.bitcast(off, jnp.uint32) < jnp.uint32(size)
            g = plsc.load_gather(buf, [off], mask=m)
            g = jnp.where(m, g, 0.0)
            plsc.addupdate(out_v.at[pl.ds(o, _L)], g)

    fire(0, 0)
    fire(0, 1)

    def field_body(i, carry):
        pltpu.sync_copy(xt_hbm.at[i], idx_v)
        wait(0)
        gather_pass(0)
        fire(i, 2)
        wait(1)
        gather_pass(1)
        fire(i, 3)
        wait(2)
        gather_pass(2)
        fire(i + 1, 0)
        wait(3)
        gather_pass(3)
        fire(i + 1, 1)
        return carry

    lax.fori_loop(0, _N_FIELDS - 1, field_body, 0)

    pltpu.sync_copy(xt_hbm.at[_N_FIELDS - 1], idx_v)
    wait(0)
    gather_pass(0)
    fire(_N_FIELDS - 1, 2)
    wait(1)
    gather_pass(1)
    fire(_N_FIELDS - 1, 3)
    wait(2)
    gather_pass(2)
    wait(3)
    gather_pass(3)

    pltpu.sync_copy(out_v, out_hbm.at[e])


def kernel(x, tables):
    xt = x.astype(jnp.int32).T
    tbl = tables.transpose(0, 2, 1).reshape(_N_FIELDS * _EMBED, _VOCAB)

    run = pl.kernel(
        _embed_body,
        out_type=jax.ShapeDtypeStruct((_EMBED, _BATCH), jnp.float32),
        mesh=plsc.VectorSubcoreMesh(core_axis_name="c", subcore_axis_name="s",
                                    num_cores=_NC, num_subcores=_NS),
        scratch_types=[
            pltpu.VMEM((_PS,), jnp.float32),
            pltpu.VMEM((_PS,), jnp.float32),
            pltpu.VMEM((_VOCAB - 3 * _PS,), jnp.float32),
            pltpu.VMEM((_BATCH,), jnp.int32),
            pltpu.VMEM((_BATCH,), jnp.float32),
            pltpu.SemaphoreType.DMA,
            pltpu.SemaphoreType.DMA,
        ],
        compiler_params=pltpu.CompilerParams(needs_layout_passes=False),
    )
    return run(xt, tbl).T


# speedup vs baseline: 1.0133x; 1.0133x over previous
import jax
import jax.numpy as jnp
from jax import lax
from jax.experimental import pallas as pl
from jax.experimental.pallas import tpu as pltpu
from jax.experimental.pallas import tpu_sc as plsc

_N_FIELDS = 26
_VOCAB = 100000
_EMBED = 32
_BATCH = 16384

_NC = 2
_NS = 16
_L = 16
_PS = 33408
_TAIL = 32
_OFFS = (0, 33408, 66560)
_MASKS = (33408, 33152, 33408)


def _embed_body(xt_hbm, tbl_hbm, tail_hbm, out_hbm, buf_a, buf_b, tail_v,
                idx_v, out_v, sem_a, sem_b):
    c = lax.axis_index("c")
    s = lax.axis_index("s")
    e = s * _NC + c

    @plsc.parallel_loop(0, _BATCH, _L, unroll=8)
    def _zero(o):
        out_v[pl.ds(o, _L)] = jnp.zeros((_L,), jnp.float32)

    bufs = (buf_a, buf_b)
    sems = (sem_a, sem_b)

    def fire(i, p, ab):
        pltpu.async_copy(
            tbl_hbm.at[i * _EMBED + e, pl.ds(_OFFS[p], _PS)],
            bufs[ab], sems[ab])

    def wait(p, ab):
        pltpu.make_async_copy(
            tbl_hbm.at[0, pl.ds(_OFFS[p], _PS)], bufs[ab], sems[ab]).wait()

    def gather_pass(p, ab):
        base = _OFFS[p]
        size = _MASKS[p]
        buf = bufs[ab]

        if p < 2:
            @plsc.parallel_loop(0, _BATCH, _L, unroll=8)
            def _gather(o):
                off = idx_v[pl.ds(o, _L)] - base
                m = plsc.bitcast(off, jnp.uint32) < jnp.uint32(size)
                g = plsc.load_gather(buf, [off], mask=m)
                g = jnp.where(m, g, 0.0)
                plsc.addupdate(out_v.at[pl.ds(o, _L)], g)
        else:
            @plsc.parallel_loop(0, _BATCH, _L, unroll=8)
            def _gather(o):
                v = idx_v[pl.ds(o, _L)]
                off = v - base
                m = plsc.bitcast(off, jnp.uint32) < jnp.uint32(size)
                g = plsc.load_gather(buf, [off], mask=m)
                g = jnp.where(m, g, 0.0)
                offt = v - (_VOCAB - _TAIL)
                mt = plsc.bitcast(offt, jnp.uint32) < jnp.uint32(_TAIL)
                gt = plsc.load_gather(tail_v, [offt], mask=mt)
                gt = jnp.where(mt, gt, 0.0)
                plsc.addupdate(out_v.at[pl.ds(o, _L)], g + gt)

    def do_field(i, first_ab, fire_plan):
        pltpu.sync_copy(xt_hbm.at[i], idx_v)
        pltpu.sync_copy(tail_hbm.at[i * _EMBED + e], tail_v)
        for p in range(3):
            ab = (first_ab + p) % 2
            wait(p, ab)
            gather_pass(p, ab)
            nxt = fire_plan[p]
            if nxt is not None:
                fire(nxt[0], nxt[1], ab)

    fire(0, 0, 0)
    fire(0, 1, 1)

    def pair_body(k, carry):
        f0 = 2 * k
        f1 = f0 + 1
        do_field(f0, 0, [(f0, 2), (f1, 0), (f1, 1)])
        do_field(f1, 1, [(f1, 2), (f1 + 1, 0), (f1 + 1, 1)])
        return carry

    lax.fori_loop(0, _N_FIELDS // 2 - 1, pair_body, 0)

    f0 = _N_FIELDS - 2
    f1 = _N_FIELDS - 1
    do_field(f0, 0, [(f0, 2), (f1, 0), (f1, 1)])
    do_field(f1, 1, [(f1, 2), None, None])

    pltpu.sync_copy(out_v, out_hbm.at[e])


def kernel(x, tables):
    xt = x.astype(jnp.int32).T
    tbl = tables.transpose(0, 2, 1).reshape(_N_FIELDS * _EMBED, _VOCAB)
    tail = lax.slice_in_dim(tbl, _VOCAB - _TAIL, _VOCAB, axis=1)

    run = pl.kernel(
        _embed_body,
        out_type=jax.ShapeDtypeStruct((_EMBED, _BATCH), jnp.float32),
        mesh=plsc.VectorSubcoreMesh(core_axis_name="c", subcore_axis_name="s",
                                    num_cores=_NC, num_subcores=_NS),
        scratch_types=[
            pltpu.VMEM((_PS,), jnp.float32),
            pltpu.VMEM((_PS,), jnp.float32),
            pltpu.VMEM((_TAIL,), jnp.float32),
            pltpu.VMEM((_BATCH,), jnp.int32),
            pltpu.VMEM((_BATCH,), jnp.float32),
            pltpu.SemaphoreType.DMA,
            pltpu.SemaphoreType.DMA,
        ],
        compiler_params=pltpu.CompilerParams(needs_layout_passes=False),
    )
    return run(xt, tbl, tail).T


# speedup vs baseline: 1.0143x; 1.0010x over previous
import jax
import jax.numpy as jnp
from jax import lax
from jax.experimental import pallas as pl
from jax.experimental.pallas import tpu as pltpu
from jax.experimental.pallas import tpu_sc as plsc

_N_FIELDS = 26
_VOCAB = 100000
_EMBED = 32
_BATCH = 16384

_NC = 2
_NS = 16
_L = 16
_PS = 33408
_TAIL = 32
_OFFS = (0, 33408, 66560)
_MASKS = (33408, 33152, 33408)


def _embed_body(xt_hbm, tbl_hbm, tail_hbm, out_hbm, buf_a, buf_b, tail_v,
                idx_v, out_v, sem_a, sem_b):
    c = lax.axis_index("c")
    s = lax.axis_index("s")
    e = s * _NC + c

    @plsc.parallel_loop(0, _BATCH, _L, unroll=8)
    def _zero(o):
        out_v[pl.ds(o, _L)] = jnp.zeros((_L,), jnp.float32)

    bufs = (buf_a, buf_b)
    sems = (sem_a, sem_b)

    def fire(i, p, ab):
        pltpu.async_copy(
            tbl_hbm.at[i * _EMBED + e, pl.ds(_OFFS[p], _PS)],
            bufs[ab], sems[ab])

    def wait(p, ab):
        pltpu.make_async_copy(
            tbl_hbm.at[0, pl.ds(_OFFS[p], _PS)], bufs[ab], sems[ab]).wait()

    def gather_pass(p, ab):
        base = _OFFS[p]
        size = _MASKS[p]
        buf = bufs[ab]

        if p < 2:
            @plsc.parallel_loop(0, _BATCH, _L, unroll=8)
            def _gather(o):
                off = idx_v[pl.ds(o, _L)] - base
                m = plsc.bitcast(off, jnp.uint32) < jnp.uint32(size)
                g = plsc.load_gather(buf, [off], mask=m)
                plsc.addupdate(out_v.at[pl.ds(o, _L)], g)
        else:
            @plsc.parallel_loop(0, _BATCH, _L, unroll=8)
            def _gather(o):
                v = idx_v[pl.ds(o, _L)]
                off = v - base
                m = plsc.bitcast(off, jnp.uint32) < jnp.uint32(size)
                g = plsc.load_gather(buf, [off], mask=m)
                offt = v - (_VOCAB - _TAIL)
                mt = plsc.bitcast(offt, jnp.uint32) < jnp.uint32(_TAIL)
                gt = plsc.load_gather(tail_v, [offt], mask=mt)
                plsc.addupdate(out_v.at[pl.ds(o, _L)], g + gt)

    def do_field(i, first_ab, fire_plan):
        pltpu.sync_copy(xt_hbm.at[i], idx_v)
        pltpu.sync_copy(tail_hbm.at[i * _EMBED + e], tail_v)
        for p in range(3):
            ab = (first_ab + p) % 2
            wait(p, ab)
            gather_pass(p, ab)
            nxt = fire_plan[p]
            if nxt is not None:
                fire(nxt[0], nxt[1], ab)

    fire(0, 0, 0)
    fire(0, 1, 1)

    def pair_body(k, carry):
        f0 = 2 * k
        f1 = f0 + 1
        do_field(f0, 0, [(f0, 2), (f1, 0), (f1, 1)])
        do_field(f1, 1, [(f1, 2), (f1 + 1, 0), (f1 + 1, 1)])
        return carry

    lax.fori_loop(0, _N_FIELDS // 2 - 1, pair_body, 0)

    f0 = _N_FIELDS - 2
    f1 = _N_FIELDS - 1
    do_field(f0, 0, [(f0, 2), (f1, 0), (f1, 1)])
    do_field(f1, 1, [(f1, 2), None, None])

    pltpu.sync_copy(out_v, out_hbm.at[e])


def kernel(x, tables):
    xt = x.astype(jnp.int32).T
    tbl = tables.transpose(0, 2, 1).reshape(_N_FIELDS * _EMBED, _VOCAB)
    tail = lax.slice_in_dim(tbl, _VOCAB - _TAIL, _VOCAB, axis=1)

    run = pl.kernel(
        _embed_body,
        out_type=jax.ShapeDtypeStruct((_EMBED, _BATCH), jnp.float32),
        mesh=plsc.VectorSubcoreMesh(core_axis_name="c", subcore_axis_name="s",
                                    num_cores=_NC, num_subcores=_NS),
        scratch_types=[
            pltpu.VMEM((_PS,), jnp.float32),
            pltpu.VMEM((_PS,), jnp.float32),
            pltpu.VMEM((_TAIL,), jnp.float32),
            pltpu.VMEM((_BATCH,), jnp.int32),
            pltpu.VMEM((_BATCH,), jnp.float32),
            pltpu.SemaphoreType.DMA,
            pltpu.SemaphoreType.DMA,
        ],
        compiler_params=pltpu.CompilerParams(needs_layout_passes=False),
    )
    return run(xt, tbl, tail).T


# speedup vs baseline: 1.1142x; 1.0984x over previous
import jax
import jax.numpy as jnp
from jax import lax
from jax.experimental import pallas as pl
from jax.experimental.pallas import tpu as pltpu
from jax.experimental.pallas import tpu_sc as plsc

_N_FIELDS = 26
_VOCAB = 100000
_EMBED = 32
_BATCH = 16384

_NC = 2
_NS = 16
_L = 16
_HALF = _BATCH // 2


def _embed_body(xt_hbm, tbl_hbm, out_hbm, row_v, idx_v, out_v, sem_r, sem_x):
    e = lax.axis_index("s") * _NC + lax.axis_index("c")

    for i in range(_N_FIELDS):
        row_cp = pltpu.async_copy(tbl_hbm.at[i * _EMBED + e], row_v, sem_r)
        for h in range(2):
            pltpu.async_copy(
                xt_hbm.at[i, pl.ds(h * _HALF, _HALF)], idx_v, sem_x).wait()
            if h == 0:
                row_cp.wait()

            if i == 0:
                @plsc.parallel_loop(0, _HALF, _L, unroll=8)
                def _first(o):
                    g = plsc.load_gather(row_v, [idx_v[pl.ds(o, _L)]])
                    out_v[pl.ds(h * _HALF + o, _L)] = g
            else:
                @plsc.parallel_loop(0, _HALF, _L, unroll=8)
                def _accum(o):
                    g = plsc.load_gather(row_v, [idx_v[pl.ds(o, _L)]])
                    plsc.addupdate(out_v.at[pl.ds(h * _HALF + o, _L)], g)

    pltpu.sync_copy(out_v, out_hbm.at[e])


def kernel(x, tables):
    xt = x.astype(jnp.int32).T
    tbl = tables.transpose(0, 2, 1).reshape(_N_FIELDS * _EMBED, _VOCAB)

    run = pl.kernel(
        _embed_body,
        out_type=jax.ShapeDtypeStruct((_EMBED, _BATCH), jnp.float32),
        mesh=plsc.VectorSubcoreMesh(core_axis_name="c", subcore_axis_name="s",
                                    num_cores=_NC, num_subcores=_NS),
        scratch_types=[
            pltpu.VMEM((_VOCAB,), jnp.float32),
            pltpu.VMEM((_HALF,), jnp.int32),
            pltpu.VMEM((_BATCH,), jnp.float32),
            pltpu.SemaphoreType.DMA,
            pltpu.SemaphoreType.DMA,
        ],
        compiler_params=pltpu.CompilerParams(needs_layout_passes=False),
    )
    return run(xt, tbl).T
